# Initial kernel scaffold; baseline (speedup 1.0000x reference)
import jax
import jax.numpy as jnp
from jax.experimental import pallas as pl


def kernel(feats, imap, omap, kernel):
    raise NotImplementedError("write your pallas kernel here")


# speedup vs baseline: 3.5859x; 3.5859x over previous
import functools

import jax
import jax.numpy as jnp
from jax import lax
from jax.experimental import pallas as pl
from jax.experimental.pallas import tpu as pltpu
from jax.experimental.pallas import tpu_sc as plsc

N = 10000
CIN = 128
COUT = 128
HALF = COUT // 2
KV = 27
CENTER = 13
PAIRS = 12000
NOFF = KV - 1

NC = 2
NS = 16

CHUNK = 125
TOTAL_PAIRS = NOFF * PAIRS
PAIRS_PER_TILE = TOTAL_PAIRS // NS
NCHUNK = PAIRS_PER_TILE // CHUNK

OUT_ROWS = 10112
ROWS_PER_TILE = OUT_ROWS // NS

MB = 1000


def _gemm_body(f_ref, w_ref, o_ref):
    res = jnp.dot(f_ref[...], w_ref[0], preferred_element_type=jnp.float32)
    o_ref[0, 0] = res[:, :HALF]
    o_ref[1, 0] = res[:, HALF:]


def _precompute_offsets(feats, weights):
    grid = (N // MB, NOFF)

    def w_index(i, k):
        return (jnp.where(k < CENTER, k, k + 1), 0, 0)

    return pl.pallas_call(
        _gemm_body,
        grid=grid,
        in_specs=[
            pl.BlockSpec((MB, CIN), lambda i, k: (i, 0)),
            pl.BlockSpec((1, CIN, COUT), w_index),
        ],
        out_specs=pl.BlockSpec((2, 1, MB, HALF), lambda i, k: (0, k, i, 0)),
        out_shape=jax.ShapeDtypeStruct((2, NOFF, N, HALF), jnp.float32),
    )(feats, weights)


def _sc_gather_scatter(p_half, gidx, sidx):
    mesh = plsc.VectorSubcoreMesh(core_axis_name="c", subcore_axis_name="s")

    @functools.partial(
        pl.kernel,
        out_type=jax.ShapeDtypeStruct((NC, OUT_ROWS, HALF), jnp.float32),
        mesh=mesh,
        compiler_params=pltpu.CompilerParams(use_tc_tiling_on_sc=False),
        scratch_types=[
            pltpu.VMEM((NCHUNK, CHUNK), jnp.int32),
            pltpu.VMEM((NCHUNK, CHUNK), jnp.int32),
            pltpu.VMEM((2, CHUNK, HALF), jnp.float32),
            pltpu.VMEM_SHARED((OUT_ROWS, HALF), jnp.float32),
            pltpu.SemaphoreType.DMA,
            pltpu.SemaphoreType.DMA,
        ],
    )
    def body(p_hbm, gidx_hbm, sidx_hbm, part_hbm, gi_v, si_v, rows_v,
             acc_sh, sem0, sem1):
        cid = lax.axis_index("c")
        sid = lax.axis_index("s")

        def zero_body(i, _):
            r = i // (HALF // 16)
            c = i % (HALF // 16)
            rows_v[0, r, pl.ds(c * 16, 16)] = jnp.zeros((16,), jnp.float32)
            return 0
        lax.fori_loop(0, CHUNK * (HALF // 16), zero_body, 0)

        base = sid * ROWS_PER_TILE
        nz = ROWS_PER_TILE // CHUNK
        for t in range(nz):
            pltpu.sync_copy(rows_v.at[0],
                            acc_sh.at[pl.ds(base + t * CHUNK, CHUNK)])
        rem = ROWS_PER_TILE - nz * CHUNK
        pltpu.sync_copy(rows_v.at[0, pl.ds(0, rem)],
                        acc_sh.at[pl.ds(base + nz * CHUNK, rem)])
        plsc.subcore_barrier()

        pltpu.sync_copy(gidx_hbm.at[sid], gi_v)
        pltpu.sync_copy(sidx_hbm.at[sid], si_v)

        sems = (sem0, sem1)

        def start(j, buf):
            pltpu.async_copy(p_hbm.at[cid].at[gi_v.at[j]], rows_v.at[buf],
                             sems[buf])

        def wait(buf):
            pltpu.make_async_copy(p_hbm.at[cid].at[gi_v.at[0]],
                                  rows_v.at[buf], sems[buf]).wait()

        def scat(j, buf):
            pltpu.sync_copy(rows_v.at[buf], acc_sh.at[si_v.at[j]], add=True)

        start(0, 0)

        def loop_body(i, _):
            j = i * 2
            start(j + 1, 1)
            wait(0)
            scat(j, 0)

            @pl.when(j + 2 < NCHUNK)
            def _():
                start(j + 2, 0)

            wait(1)
            scat(j + 1, 1)
            return 0

        lax.fori_loop(0, NCHUNK // 2, loop_body, 0)

        plsc.subcore_barrier()
        pltpu.sync_copy(acc_sh.at[pl.ds(base, ROWS_PER_TILE)],
                        part_hbm.at[cid, pl.ds(base, ROWS_PER_TILE)])

    return body(p_half, gidx, sidx)


def _finish_body(f_ref, w_ref, p0_ref, p1_ref, o_ref):
    res = jnp.dot(f_ref[...], w_ref[0], preferred_element_type=jnp.float32)
    o_ref[...] = res + jnp.concatenate([p0_ref[0], p1_ref[0]], axis=1)


def _finish(feats, weights, part):
    grid = (N // MB,)
    return pl.pallas_call(
        _finish_body,
        grid=grid,
        in_specs=[
            pl.BlockSpec((MB, CIN), lambda i: (i, 0)),
            pl.BlockSpec((1, CIN, COUT), lambda i: (CENTER, 0, 0)),
            pl.BlockSpec((1, MB, HALF), lambda i: (0, i, 0)),
            pl.BlockSpec((1, MB, HALF), lambda i: (1, i, 0)),
        ],
        out_specs=pl.BlockSpec((MB, COUT), lambda i: (i, 0)),
        out_shape=jax.ShapeDtypeStruct((N, COUT), jnp.float32),
    )(feats, weights, part, part)


def kernel(feats, imap, omap, kernel):
    weights = kernel

    off = jnp.arange(NOFF, dtype=jnp.int32) * N
    gidx = (imap + off[:, None]).reshape(NS, NCHUNK, CHUNK)
    sidx = omap.reshape(NS, NCHUNK, CHUNK)

    p_half = _precompute_offsets(feats, weights)
    p_flat = p_half.reshape(2, NOFF * N, HALF)
    part = _sc_gather_scatter(p_flat, gidx, sidx)
    return _finish(feats, weights, part)
